# baseline (speedup 1.0000x reference)
import jax
import jax.numpy as jnp
from jax import lax
from jax.experimental import pallas as pl
from jax.experimental.pallas import tpu as pltpu
from jax.experimental.pallas import tpu_sc as plsc

NC, NS, L = 2, 16, 16
NW = NC * NS
B = 16384
D = 128
BPW = B // NW
CH = 128
NCHUNK = BPW // CH
NSLOT = 2
GP = CH // L
PST = L + 1


def _tree_sum(vals):
    vals = list(vals)
    while len(vals) > 1:
        nxt = [a + b for a, b in zip(vals[0::2], vals[1::2])]
        if len(vals) % 2:
            nxt.append(vals[-1])
        vals = nxt
    return vals[0]


def _sc_body(vi_hbm, vj_hbm, a_hbm, b_hbm, out_hbm,
             idxa_v, idxb_v, rows_a, rows_b, pbuf, out_v, sems, isem):
    wid = lax.axis_index("s") * NC + lax.axis_index("c")
    base = wid * BPW
    lanes = lax.iota(jnp.int32, L)

    cpa = pltpu.async_copy(vi_hbm.at[pl.ds(base, BPW)], idxa_v, isem)
    cpb = pltpu.async_copy(vj_hbm.at[pl.ds(base, BPW)], idxb_v, isem)
    cpa.wait()
    cpb.wait()

    def start(ch, slot):
        ia = idxa_v.at[pl.ds(ch * CH, CH)]
        ib = idxb_v.at[pl.ds(ch * CH, CH)]
        pltpu.async_copy(a_hbm.at[ia], rows_a.at[slot], sems.at[slot])
        pltpu.async_copy(b_hbm.at[ib], rows_b.at[slot], sems.at[slot])

    def drain(slot):
        pltpu.make_async_copy(a_hbm.at[idxa_v.at[pl.ds(0, CH)]],
                              rows_a.at[slot], sems.at[slot]).wait()
        pltpu.make_async_copy(b_hbm.at[idxb_v.at[pl.ds(0, CH)]],
                              rows_b.at[slot], sems.at[slot]).wait()

    for p in range(NSLOT - 1):
        start(p, p)

    def chunk(ch, carry):
        slot = lax.rem(ch, NSLOT)

        @pl.when(ch + NSLOT - 1 < NCHUNK)
        def _():
            start(ch + NSLOT - 1, lax.rem(ch + NSLOT - 1, NSLOT))

        drain(slot)
        ra = rows_a.at[slot]
        rb = rows_b.at[slot]

        @plsc.parallel_loop(0, GP, unroll=4)
        def group(g):
            pb = g * (L * PST)
            for r in range(L):
                row = g * L + r
                acc = _tree_sum(ra[row, pl.ds(e * L, L)] * rb[row, pl.ds(e * L, L)]
                                for e in range(D // L))
                pbuf[pl.ds(pb + r * PST, L)] = acc
            csum = _tree_sum(plsc.load_gather(pbuf, [pb + lanes * PST + l])
                             for l in range(L))
            out_v[pl.ds(ch * CH + g * L, L)] = csum

        return carry

    lax.fori_loop(0, NCHUNK, chunk, 0)
    pltpu.sync_copy(out_v, out_hbm.at[pl.ds(base, BPW)])


def kernel(v_i, v_j, first_emb, second_emb, context_emb):
    del first_emb
    mesh = plsc.VectorSubcoreMesh(core_axis_name="c", subcore_axis_name="s",
                                  num_cores=NC, num_subcores=NS)
    f = pl.kernel(
        _sc_body,
        out_type=jax.ShapeDtypeStruct((B,), jnp.float32),
        mesh=mesh,
        compiler_params=pltpu.CompilerParams(
            needs_layout_passes=False,
            disable_bounds_checks=True,
            disable_semaphore_checks=True,
            skip_device_barrier=True,
        ),
        scratch_types=[
            pltpu.VMEM((BPW,), jnp.int32),
            pltpu.VMEM((BPW,), jnp.int32),
            pltpu.VMEM((NSLOT, CH, D), jnp.float32),
            pltpu.VMEM((NSLOT, CH, D), jnp.float32),
            pltpu.VMEM((GP * L * PST,), jnp.float32),
            pltpu.VMEM((BPW,), jnp.float32),
            pltpu.SemaphoreType.DMA((NSLOT,)),
            pltpu.SemaphoreType.DMA,
        ],
    )
    return f(v_i.astype(jnp.int32), v_j.astype(jnp.int32),
             second_emb, context_emb)


# speedup vs baseline: 1.0459x; 1.0459x over previous
import jax
import jax.numpy as jnp
from jax import lax
from jax.experimental import pallas as pl
from jax.experimental.pallas import tpu as pltpu
from jax.experimental.pallas import tpu_sc as plsc

NC, NS, L = 2, 16, 16
NW = NC * NS
B = 16384
D = 128
BPW = B // NW
CH = 128
NCHUNK = BPW // CH
NSLOT = 2
GP = CH // L
PST = L + 1


def _tree_sum(vals):
    vals = list(vals)
    while len(vals) > 1:
        nxt = [a + b for a, b in zip(vals[0::2], vals[1::2])]
        if len(vals) % 2:
            nxt.append(vals[-1])
        vals = nxt
    return vals[0]


def _sc_body(vi_hbm, vj_hbm, a_hbm, b_hbm, out_hbm,
             idxa_v, idxb_v, rows_a, rows_b, pbuf, out_v, sems, isem):
    wid = lax.axis_index("s") * NC + lax.axis_index("c")
    base = wid * BPW
    lanes = lax.iota(jnp.int32, L)

    cpa = pltpu.async_copy(vi_hbm.at[pl.ds(base, BPW)], idxa_v, isem)
    cpb = pltpu.async_copy(vj_hbm.at[pl.ds(base, BPW)], idxb_v, isem)
    cpa.wait()
    cpb.wait()

    def start(ch, slot):
        ia = idxa_v.at[pl.ds(ch * CH, CH)]
        ib = idxb_v.at[pl.ds(ch * CH, CH)]
        pltpu.async_copy(a_hbm.at[ia], rows_a.at[slot], sems.at[slot])
        pltpu.async_copy(b_hbm.at[ib], rows_b.at[slot], sems.at[slot])

    def drain(slot):
        pltpu.make_async_copy(a_hbm.at[idxa_v.at[pl.ds(0, CH)]],
                              rows_a.at[slot], sems.at[slot]).wait()
        pltpu.make_async_copy(b_hbm.at[idxb_v.at[pl.ds(0, CH)]],
                              rows_b.at[slot], sems.at[slot]).wait()

    for p in range(NSLOT - 1):
        start(p, p)

    def chunk(ch, carry):
        slot = lax.rem(ch, NSLOT)

        @pl.when(ch + NSLOT - 1 < NCHUNK)
        def _():
            start(ch + NSLOT - 1, lax.rem(ch + NSLOT - 1, NSLOT))

        drain(slot)
        ra = rows_a.at[slot]
        rb = rows_b.at[slot]

        @plsc.parallel_loop(0, GP, unroll=2)
        def group(g):
            pb = g * (L * PST)
            for r in range(L):
                row = g * L + r
                acc = _tree_sum(ra[row, pl.ds(e * L, L)] * rb[row, pl.ds(e * L, L)]
                                for e in range(D // L))
                pbuf[pl.ds(pb + r * PST, L)] = acc
            csum = _tree_sum(plsc.load_gather(pbuf, [pb + lanes * PST + l])
                             for l in range(L))
            out_v[pl.ds(ch * CH + g * L, L)] = csum

        return carry

    lax.fori_loop(0, NCHUNK, chunk, 0)
    pltpu.sync_copy(out_v, out_hbm.at[pl.ds(base, BPW)])


def kernel(v_i, v_j, first_emb, second_emb, context_emb):
    del first_emb
    mesh = plsc.VectorSubcoreMesh(core_axis_name="c", subcore_axis_name="s",
                                  num_cores=NC, num_subcores=NS)
    f = pl.kernel(
        _sc_body,
        out_type=jax.ShapeDtypeStruct((B,), jnp.float32),
        mesh=mesh,
        compiler_params=pltpu.CompilerParams(
            needs_layout_passes=False,
            disable_bounds_checks=True,
            disable_semaphore_checks=True,
            skip_device_barrier=True,
        ),
        scratch_types=[
            pltpu.VMEM((BPW,), jnp.int32),
            pltpu.VMEM((BPW,), jnp.int32),
            pltpu.VMEM((NSLOT, CH, D), jnp.float32),
            pltpu.VMEM((NSLOT, CH, D), jnp.float32),
            pltpu.VMEM((GP * L * PST,), jnp.float32),
            pltpu.VMEM((BPW,), jnp.float32),
            pltpu.SemaphoreType.DMA((NSLOT,)),
            pltpu.SemaphoreType.DMA,
        ],
    )
    return f(v_i.astype(jnp.int32), v_j.astype(jnp.int32),
             second_emb, context_emb)
